# initial kernel scaffold (unmeasured)
import jax
import jax.numpy as jnp
from jax import lax
from jax.experimental import pallas as pl
from jax.experimental.pallas import tpu as pltpu


def kernel(
    x,
):
    def body(*refs):
        pass

    out_shape = jax.ShapeDtypeStruct(..., jnp.float32)
    return pl.pallas_call(body, out_shape=out_shape)(...)


# baseline (device time: 47159 ns/iter reference)
import jax
import jax.numpy as jnp
from jax import lax
from jax.experimental import pallas as pl
from jax.experimental.pallas import tpu as pltpu

N_DEV = 8
K = 16
INT_MIN = jnp.iinfo(jnp.int32).min


def _extract_topk_desc(keys, k):
    outs = []
    m = jnp.max(keys, axis=1)
    outs.append(m)
    for _ in range(k - 1):
        masked = jnp.where(keys < m[:, None], keys, INT_MIN)
        m = jnp.max(masked, axis=1)
        outs.append(m)
    return jnp.stack(outs, axis=1)


def kernel(x):
    m_rows, n_cols = x.shape

    def body(x_ref, out_ref, gather_ref, send_sems, recv_sems):
        my_pos = lax.axis_index("i")

        xv = x_ref[...]
        vbits = lax.bitcast_convert_type(
            xv.astype(jnp.bfloat16).astype(jnp.float32), jnp.int32
        )
        col = lax.broadcasted_iota(jnp.int32, xv.shape, 1)
        keys = vbits | (my_pos * n_cols + col)
        local_top = _extract_topk_desc(keys, K)

        gather_ref[pl.ds(my_pos, 1)] = local_top[None]

        barrier_sem = pltpu.get_barrier_semaphore()
        for off in range(1, N_DEV):
            p = lax.rem(my_pos + off, N_DEV)
            pl.semaphore_signal(
                barrier_sem, inc=1,
                device_id=(p,), device_id_type=pl.DeviceIdType.MESH,
            )
        pl.semaphore_wait(barrier_sem, N_DEV - 1)

        sends = []
        for off in range(1, N_DEV):
            p = lax.rem(my_pos + off, N_DEV)
            rdma = pltpu.make_async_remote_copy(
                src_ref=gather_ref.at[my_pos],
                dst_ref=gather_ref.at[my_pos],
                send_sem=send_sems.at[off],
                recv_sem=recv_sems.at[my_pos],
                device_id=(p,),
                device_id_type=pl.DeviceIdType.MESH,
            )
            rdma.start()
            sends.append(rdma)

        for off in range(1, N_DEV):
            s = lax.rem(my_pos + off, N_DEV)
            recv = pltpu.make_async_remote_copy(
                src_ref=gather_ref.at[my_pos],
                dst_ref=gather_ref.at[s],
                send_sem=send_sems.at[off],
                recv_sem=recv_sems.at[s],
                device_id=(s,),
                device_id_type=pl.DeviceIdType.MESH,
            )
            recv.wait_recv()

        g = gather_ref[...]
        cand = jnp.concatenate([g[s] for s in range(N_DEV)], axis=1)
        final = _extract_topk_desc(cand, K)
        out_ref[...] = lax.bitcast_convert_type(
            final & jnp.int32(-65536), jnp.float32
        )

        for rdma in sends:
            rdma.wait_send()

    return pl.pallas_call(
        body,
        out_shape=jax.ShapeDtypeStruct((m_rows, K), jnp.float32),
        in_specs=[pl.BlockSpec(memory_space=pltpu.VMEM)],
        out_specs=pl.BlockSpec(memory_space=pltpu.VMEM),
        scratch_shapes=[
            pltpu.VMEM((N_DEV, m_rows, K), jnp.int32),
            pltpu.SemaphoreType.DMA((N_DEV,)),
            pltpu.SemaphoreType.DMA((N_DEV,)),
        ],
        compiler_params=pltpu.CompilerParams(collective_id=0),
    )(x)
